# baseline (speedup 1.0000x reference)
import math

import jax
import jax.numpy as jnp
from jax.experimental import pallas as pl
from jax.experimental.pallas import tpu as pltpu

_EXTENTS = 0.2
_RADIUS = _EXTENTS / 2.0
_KNN = 64


def _conv_body(blo_ref, bhi_ref, clo_ref, chi_ref,
               pos_col_ref, pos_row_ref, feat_ref, wc_ref, b_ref, out_ref,
               d2_ref, theta_ref, acc_ref, *, jc):
    rb = pos_col_ref.shape[1]
    n = pos_row_ref.shape[2]
    nc = n // jc
    f32 = jnp.float32

    ti = pl.program_id(0)
    bi = pl.program_id(1)

    pc = pos_col_ref[0]
    px_c = pc[:, 0:1]
    py_c = pc[:, 1:2]
    pz_c = pc[:, 2:3]
    pr = pos_row_ref[0]

    r2 = f32(_RADIUS * _RADIUS)
    rm = f32(_RADIUS * (1.0 + 1e-5) + 1e-6)
    my_lo = blo_ref[ti, bi]
    my_hi = bhi_ref[ti, bi]
    hid = wc_ref.shape[1]
    inv_r = f32(1.0 / _RADIUS)
    bf16 = jnp.bfloat16

    def chunk_active(c):
        return jnp.logical_and(clo_ref[ti, c] - my_hi <= rm,
                               my_lo - chi_ref[ti, c] <= rm)

    in_dim = feat_ref.shape[2]

    def accumulate(theta):
        acc_ref[...] = jnp.zeros((rb, 27 * in_dim), dtype=f32)
        theta_ref[:, 1:2] = jnp.zeros((rb, 1), dtype=f32)
        for c in range(nc):
            @pl.when(chunk_active(c))
            def _(c=c):
                sl = slice(c * jc, (c + 1) * jc)
                dx = pr[0:1, sl] - px_c
                dy = pr[1:2, sl] - py_c
                dz = pr[2:3, sl] - pz_c
                d2 = dx * dx + dy * dy + dz * dz
                mask = (d2 <= theta).astype(f32)
                theta_ref[:, 1:2] += jnp.sum(mask, axis=1, keepdims=True)
                q = d2 * f32(1.0 / (_RADIUS * _RADIUS))
                norm2 = jnp.sqrt(q + f32(1e-12))
                ninf = jnp.maximum(jnp.maximum(jnp.abs(dx), jnp.abs(dy)),
                                   jnp.abs(dz)) * inv_r
                scale = jnp.where(ninf > f32(1e-8),
                                  norm2 / jnp.maximum(ninf, f32(1e-8)), f32(0.0))
                scale_d = scale * inv_r

                def axis_w(dv, m):
                    u = jnp.clip(dv * scale_d, f32(-1.0), f32(1.0))
                    whi = jnp.maximum(u, f32(0.0))
                    wlo = whi - u
                    wmid = (f32(1.0) - whi) - wlo
                    if m is not None:
                        wlo, wmid, whi = wlo * m, wmid * m, whi * m
                    return wlo.astype(bf16), wmid.astype(bf16), whi.astype(bf16)

                wx = axis_w(dx, mask)
                wy = axis_w(dy, None)
                wz = axis_w(dz, None)
                feat_c = feat_ref[0, sl, :]
                for a in range(3):
                    for bb in range(3):
                        pab = wx[a] * wy[bb]
                        for cc in range(3):
                            tap = a * 9 + bb * 3 + cc
                            plane = pab * wz[cc]
                            agg = jnp.dot(plane, feat_c,
                                          preferred_element_type=f32)
                            acc_ref[:, tap * in_dim:(tap + 1) * in_dim] += agg

    accumulate(r2)

    @pl.when(jnp.max(theta_ref[:, 1:2]) > f32(_KNN))
    def _cap():
        d2_ref[...] = jnp.ones((rb, n), dtype=f32)
        for c in range(nc):
            @pl.when(chunk_active(c))
            def _(c=c):
                sl = slice(c * jc, (c + 1) * jc)
                dx = pr[0:1, sl] - px_c
                dy = pr[1:2, sl] - py_c
                dz = pr[2:3, sl] - pz_c
                d2_ref[:, sl] = dx * dx + dy * dy + dz * dz
        d2 = d2_ref[...]
        cnt_r = jnp.sum((d2 <= r2).astype(f32), axis=1, keepdims=True)
        lo = jnp.zeros((rb, 1), dtype=f32)
        hi = jnp.full((rb, 1), r2, dtype=f32)
        for _ in range(35):
            mid = (lo + hi) * f32(0.5)
            cm = jnp.sum((d2 <= mid).astype(f32), axis=1, keepdims=True)
            ge = cm >= f32(_KNN)
            hi = jnp.where(ge, mid, hi)
            lo = jnp.where(ge, lo, mid)
        theta_ref[:, 0:1] = jnp.where(cnt_r > f32(_KNN), hi, r2)
        accumulate(theta_ref[:, 0:1])

    cnt = jnp.maximum(theta_ref[:, 1:2], f32(1.0))
    acc = jnp.dot(acc_ref[...].astype(bf16), wc_ref[...],
                  preferred_element_type=f32)
    out = acc / cnt + b_ref[0:1, :]
    out_ref[0] = jnp.maximum(out, f32(0.0))


def _tfm_body(x_ref, pe_ref, wq_ref, bq_ref, wkv_ref, bkv_ref, wo_ref, bo_ref,
              ln1g_ref, ln1b_ref, ffw1_ref, ffb1_ref, ffw2_ref, ffb2_ref,
              ln2g_ref, ln2b_ref, fcw_ref, fcb_ref, out_ref, *, heads):
    f32 = jnp.float32
    t, nb, h = x_ref.shape
    hd = h // heads

    x = x_ref[...] + pe_ref[...]
    x3 = x[t - 1]
    q3 = jnp.dot(x3, wq_ref[...], preferred_element_type=f32) + bq_ref[0:1, :]
    xf = x.reshape(t * nb, h)
    kv = jnp.dot(xf, wkv_ref[...], preferred_element_type=f32) + bkv_ref[0:1, :]
    k = kv[:, :h].reshape(t, nb, h)
    v = kv[:, h:].reshape(t, nb, h)

    ri = jax.lax.broadcasted_iota(jnp.int32, (h, heads), 0)
    ci = jax.lax.broadcasted_iota(jnp.int32, (h, heads), 1)
    seg = (ri // hd == ci).astype(f32)

    inv_sqrt_hd = f32(1.0 / math.sqrt(hd))
    logits = [jnp.dot(q3 * k[t2], seg, preferred_element_type=f32) * inv_sqrt_hd
              for t2 in range(t)]
    m = logits[0]
    for t2 in range(1, t):
        m = jnp.maximum(m, logits[t2])
    exps = [jnp.exp(l - m) for l in logits]
    ssum = exps[0]
    for t2 in range(1, t):
        ssum = ssum + exps[t2]
    o3 = jnp.zeros((nb, h), dtype=f32)
    for t2 in range(t):
        p = exps[t2] / ssum
        pe = jnp.dot(p, seg.T, preferred_element_type=f32)
        o3 = o3 + pe * v[t2]

    a3 = jnp.dot(o3, wo_ref[...], preferred_element_type=f32) + bo_ref[0:1, :]

    def ln(y, g_ref, b_ref):
        mu = jnp.mean(y, axis=1, keepdims=True)
        d = y - mu
        var = jnp.mean(d * d, axis=1, keepdims=True)
        return d * jax.lax.rsqrt(var + f32(1e-5)) * g_ref[0:1, :] + b_ref[0:1, :]

    h1 = ln(x3 + a3, ln1g_ref, ln1b_ref)
    ffh = jnp.maximum(jnp.dot(h1, ffw1_ref[...], preferred_element_type=f32)
                      + ffb1_ref[0:1, :], f32(0.0))
    ff2 = jnp.dot(ffh, ffw2_ref[...], preferred_element_type=f32) + ffb2_ref[0:1, :]
    h2 = ln(h1 + ff2, ln2g_ref, ln2b_ref)
    out_ref[...] = jnp.dot(h2, fcw_ref[...], preferred_element_type=f32) + fcb_ref[0:1, :]


def _posenc(n, d):
    pos = jnp.arange(n, dtype=jnp.float32)[:, None]
    div = jnp.exp(jnp.arange(0, d, 2, dtype=jnp.float32) * (-math.log(10000.0) / d))
    enc = jnp.zeros((n, d), jnp.float32)
    enc = enc.at[:, 0::2].set(jnp.sin(pos * div))
    enc = enc.at[:, 1::2].set(jnp.cos(pos * div))
    return enc


def kernel(features, W_conv, b_conv, in_proj_w, in_proj_b, out_proj_w, out_proj_b,
           ln1_g, ln1_b, ff_w1, ff_b1, ff_w2, ff_b2, ln2_g, ln2_b, fc_w, fc_b):
    t, n, in_dim = features.shape
    hid = W_conv.shape[2]
    out_dim = fc_w.shape[0]
    heads = 4

    order = jnp.argsort(features[:, :, 0], axis=1).astype(jnp.int32)
    inv_order = jnp.argsort(order, axis=1).astype(jnp.int32)
    feat_s = jnp.take_along_axis(features.astype(jnp.bfloat16),
                                 order[:, :, None], axis=1)
    pos_s = jnp.take_along_axis(features[:, :, :3], order[:, :, None], axis=1)
    pos_col = jnp.pad(pos_s, ((0, 0), (0, 0), (0, 128 - 3)))
    pos_row = jnp.transpose(pos_s, (0, 2, 1))
    pe = _posenc(n, hid)
    b2d = b_conv[None, :]

    rb = min(128, n)
    nblk = n // rb
    jc = min(256, n)
    nc = n // jc
    xs = pos_s[:, :, 0]
    blo = xs[:, ::rb]
    bhi = xs[:, rb - 1::rb]
    clo = xs[:, ::jc]
    chi = xs[:, jc - 1::jc]

    import functools
    smem = pl.BlockSpec(memory_space=pltpu.SMEM)

    x_s = pl.pallas_call(
        functools.partial(_conv_body, jc=jc),
        grid=(t, nblk),
        in_specs=[
            smem, smem, smem, smem,
            pl.BlockSpec((1, rb, 128), lambda ti, bi: (ti, bi, 0)),
            pl.BlockSpec((1, 3, n), lambda ti, bi: (ti, 0, 0)),
            pl.BlockSpec((1, n, in_dim), lambda ti, bi: (ti, 0, 0)),
            pl.BlockSpec((27 * in_dim, hid), lambda ti, bi: (0, 0)),
            pl.BlockSpec((1, hid), lambda ti, bi: (0, 0)),
        ],
        out_specs=pl.BlockSpec((1, rb, hid), lambda ti, bi: (ti, bi, 0)),
        out_shape=jax.ShapeDtypeStruct((t, n, hid), jnp.float32),
        scratch_shapes=[pltpu.VMEM((rb, n), jnp.float32),
                        pltpu.VMEM((rb, 128), jnp.float32),
                        pltpu.VMEM((rb, 27 * in_dim), jnp.float32)],
        compiler_params=pltpu.CompilerParams(
            dimension_semantics=("parallel", "parallel")),
    )(blo, bhi, clo, chi, pos_col, pos_row, feat_s,
      W_conv.reshape(27 * in_dim, hid).astype(jnp.bfloat16), b2d)

    x = jnp.take_along_axis(x_s, inv_order[:, :, None], axis=1)

    nb = min(256, n)
    nblk2 = n // nb
    wq_t = in_proj_w[:hid].T
    wkv_t = in_proj_w[hid:].T
    bq = in_proj_b[None, :hid]
    bkv = in_proj_b[None, hid:]

    body = functools.partial(_tfm_body, heads=heads)

    y = pl.pallas_call(
        body,
        grid=(nblk2,),
        in_specs=[
            pl.BlockSpec((t, nb, hid), lambda bi: (0, bi, 0)),
            pl.BlockSpec((nb, hid), lambda bi: (bi, 0)),
            pl.BlockSpec((hid, hid), lambda bi: (0, 0)),
            pl.BlockSpec((1, hid), lambda bi: (0, 0)),
            pl.BlockSpec((hid, 2 * hid), lambda bi: (0, 0)),
            pl.BlockSpec((1, 2 * hid), lambda bi: (0, 0)),
            pl.BlockSpec((hid, hid), lambda bi: (0, 0)),
            pl.BlockSpec((1, hid), lambda bi: (0, 0)),
            pl.BlockSpec((1, hid), lambda bi: (0, 0)),
            pl.BlockSpec((1, hid), lambda bi: (0, 0)),
            pl.BlockSpec((hid, 4 * hid), lambda bi: (0, 0)),
            pl.BlockSpec((1, 4 * hid), lambda bi: (0, 0)),
            pl.BlockSpec((4 * hid, hid), lambda bi: (0, 0)),
            pl.BlockSpec((1, hid), lambda bi: (0, 0)),
            pl.BlockSpec((1, hid), lambda bi: (0, 0)),
            pl.BlockSpec((1, hid), lambda bi: (0, 0)),
            pl.BlockSpec((hid, out_dim), lambda bi: (0, 0)),
            pl.BlockSpec((1, out_dim), lambda bi: (0, 0)),
        ],
        out_specs=pl.BlockSpec((nb, out_dim), lambda bi: (bi, 0)),
        out_shape=jax.ShapeDtypeStruct((n, out_dim), jnp.float32),
        compiler_params=pltpu.CompilerParams(
            dimension_semantics=("parallel",)),
    )(x, pe, wq_t, bq, wkv_t, bkv, out_proj_w.T, out_proj_b[None, :],
      ln1_g[None, :], ln1_b[None, :], ff_w1.T, ff_b1[None, :],
      ff_w2.T, ff_b2[None, :], ln2_g[None, :], ln2_b[None, :],
      fc_w.T, fc_b[None, :])

    return y


# speedup vs baseline: 1.1643x; 1.1643x over previous
import math

import jax
import jax.numpy as jnp
from jax.experimental import pallas as pl
from jax.experimental.pallas import tpu as pltpu

_EXTENTS = 0.2
_RADIUS = _EXTENTS / 2.0
_KNN = 64


def _conv_body(blo_ref, bhi_ref, clo_ref, chi_ref,
               pos_col_ref, pos_row_ref, feat_ref, wc_ref, b_ref, out_ref,
               d2_ref, theta_ref, acc_ref, *, jc):
    rb = pos_col_ref.shape[1]
    n = pos_row_ref.shape[2]
    nc = n // jc
    f32 = jnp.float32

    ti = pl.program_id(0)
    bi = pl.program_id(1)

    pc = pos_col_ref[0]
    px_c = pc[:, 0:1]
    py_c = pc[:, 1:2]
    pz_c = pc[:, 2:3]
    pr = pos_row_ref[0]

    r2 = f32(_RADIUS * _RADIUS)
    rm = f32(_RADIUS * (1.0 + 1e-5) + 1e-6)
    my_lo = blo_ref[ti, bi]
    my_hi = bhi_ref[ti, bi]
    hid = wc_ref.shape[1]
    inv_r = f32(1.0 / _RADIUS)
    bf16 = jnp.bfloat16

    def chunk_active(c):
        return jnp.logical_and(clo_ref[ti, c] - my_hi <= rm,
                               my_lo - chi_ref[ti, c] <= rm)

    in_dim = feat_ref.shape[2]

    def accumulate(theta):
        acc_ref[...] = jnp.zeros((rb, 27 * in_dim), dtype=f32)
        theta_ref[:, 1:2] = jnp.zeros((rb, 1), dtype=f32)
        for c in range(nc):
            @pl.when(chunk_active(c))
            def _(c=c):
                sl = slice(c * jc, (c + 1) * jc)
                dx = pr[0:1, sl] - px_c
                dy = pr[1:2, sl] - py_c
                dz = pr[2:3, sl] - pz_c
                d2 = dx * dx + dy * dy + dz * dz
                mask = (d2 <= theta).astype(f32)
                theta_ref[:, 1:2] += jnp.sum(mask, axis=1, keepdims=True)
                q = d2 * f32(1.0 / (_RADIUS * _RADIUS))
                norm2 = jnp.sqrt(q + f32(1e-12))
                ninf = jnp.maximum(jnp.maximum(jnp.abs(dx), jnp.abs(dy)),
                                   jnp.abs(dz)) * inv_r
                scale = jnp.where(ninf > f32(1e-8),
                                  norm2 / jnp.maximum(ninf, f32(1e-8)), f32(0.0))
                scale_d = scale * inv_r

                def axis_w(dv, m):
                    u = jnp.clip(dv * scale_d, f32(-1.0), f32(1.0))
                    whi = jnp.maximum(u, f32(0.0))
                    wlo = whi - u
                    wmid = (f32(1.0) - whi) - wlo
                    if m is not None:
                        wlo, wmid, whi = wlo * m, wmid * m, whi * m
                    return wlo.astype(bf16), wmid.astype(bf16), whi.astype(bf16)

                wx = axis_w(dx, mask)
                wy = axis_w(dy, None)
                wz = axis_w(dz, None)
                feat_c = feat_ref[0, sl, :]
                for a in range(3):
                    for bb in range(3):
                        pab = wx[a] * wy[bb]
                        for cc in range(3):
                            tap = a * 9 + bb * 3 + cc
                            plane = pab * wz[cc]
                            agg = jnp.dot(plane, feat_c,
                                          preferred_element_type=f32)
                            acc_ref[:, tap * in_dim:(tap + 1) * in_dim] += agg

    accumulate(r2)

    @pl.when(jnp.max(theta_ref[:, 1:2]) > f32(_KNN))
    def _cap():
        d2_ref[...] = jnp.ones((rb, n), dtype=f32)
        for c in range(nc):
            @pl.when(chunk_active(c))
            def _(c=c):
                sl = slice(c * jc, (c + 1) * jc)
                dx = pr[0:1, sl] - px_c
                dy = pr[1:2, sl] - py_c
                dz = pr[2:3, sl] - pz_c
                d2_ref[:, sl] = dx * dx + dy * dy + dz * dz
        d2 = d2_ref[...]
        cnt_r = jnp.sum((d2 <= r2).astype(f32), axis=1, keepdims=True)
        lo = jnp.zeros((rb, 1), dtype=f32)
        hi = jnp.full((rb, 1), r2, dtype=f32)
        for _ in range(35):
            mid = (lo + hi) * f32(0.5)
            cm = jnp.sum((d2 <= mid).astype(f32), axis=1, keepdims=True)
            ge = cm >= f32(_KNN)
            hi = jnp.where(ge, mid, hi)
            lo = jnp.where(ge, lo, mid)
        theta_ref[:, 0:1] = jnp.where(cnt_r > f32(_KNN), hi, r2)
        accumulate(theta_ref[:, 0:1])

    cnt = jnp.maximum(theta_ref[:, 1:2], f32(1.0))
    acc = jnp.dot(acc_ref[...].astype(bf16), wc_ref[...],
                  preferred_element_type=f32)
    out = acc / cnt + b_ref[0:1, :]
    out_ref[0] = jnp.maximum(out, f32(0.0))


def _tfm_body(x_ref, pe_ref, wq_ref, bq_ref, wkv_ref, bkv_ref, wo_ref, bo_ref,
              ln1g_ref, ln1b_ref, ffw1_ref, ffb1_ref, ffw2_ref, ffb2_ref,
              ln2g_ref, ln2b_ref, fcw_ref, fcb_ref, out_ref, *, heads):
    f32 = jnp.float32
    t, nb, h = x_ref.shape
    hd = h // heads

    x = x_ref[...] + pe_ref[...]
    x3 = x[t - 1]
    q3 = jnp.dot(x3, wq_ref[...], preferred_element_type=f32) + bq_ref[0:1, :]
    xf = x.reshape(t * nb, h)
    kv = jnp.dot(xf, wkv_ref[...], preferred_element_type=f32) + bkv_ref[0:1, :]
    k = kv[:, :h].reshape(t, nb, h)
    v = kv[:, h:].reshape(t, nb, h)

    ri = jax.lax.broadcasted_iota(jnp.int32, (h, heads), 0)
    ci = jax.lax.broadcasted_iota(jnp.int32, (h, heads), 1)
    seg = (ri // hd == ci).astype(f32)

    inv_sqrt_hd = f32(1.0 / math.sqrt(hd))
    logits = [jnp.dot(q3 * k[t2], seg, preferred_element_type=f32) * inv_sqrt_hd
              for t2 in range(t)]
    m = logits[0]
    for t2 in range(1, t):
        m = jnp.maximum(m, logits[t2])
    exps = [jnp.exp(l - m) for l in logits]
    ssum = exps[0]
    for t2 in range(1, t):
        ssum = ssum + exps[t2]
    o3 = jnp.zeros((nb, h), dtype=f32)
    for t2 in range(t):
        p = exps[t2] / ssum
        pe = jnp.dot(p, seg.T, preferred_element_type=f32)
        o3 = o3 + pe * v[t2]

    a3 = jnp.dot(o3, wo_ref[...], preferred_element_type=f32) + bo_ref[0:1, :]

    def ln(y, g_ref, b_ref):
        mu = jnp.mean(y, axis=1, keepdims=True)
        d = y - mu
        var = jnp.mean(d * d, axis=1, keepdims=True)
        return d * jax.lax.rsqrt(var + f32(1e-5)) * g_ref[0:1, :] + b_ref[0:1, :]

    h1 = ln(x3 + a3, ln1g_ref, ln1b_ref)
    ffh = jnp.maximum(jnp.dot(h1, ffw1_ref[...], preferred_element_type=f32)
                      + ffb1_ref[0:1, :], f32(0.0))
    ff2 = jnp.dot(ffh, ffw2_ref[...], preferred_element_type=f32) + ffb2_ref[0:1, :]
    h2 = ln(h1 + ff2, ln2g_ref, ln2b_ref)
    out_ref[...] = jnp.dot(h2, fcw_ref[...], preferred_element_type=f32) + fcb_ref[0:1, :]


def _posenc(n, d):
    pos = jnp.arange(n, dtype=jnp.float32)[:, None]
    div = jnp.exp(jnp.arange(0, d, 2, dtype=jnp.float32) * (-math.log(10000.0) / d))
    enc = jnp.zeros((n, d), jnp.float32)
    enc = enc.at[:, 0::2].set(jnp.sin(pos * div))
    enc = enc.at[:, 1::2].set(jnp.cos(pos * div))
    return enc


def kernel(features, W_conv, b_conv, in_proj_w, in_proj_b, out_proj_w, out_proj_b,
           ln1_g, ln1_b, ff_w1, ff_b1, ff_w2, ff_b2, ln2_g, ln2_b, fc_w, fc_b):
    t, n, in_dim = features.shape
    hid = W_conv.shape[2]
    out_dim = fc_w.shape[0]
    heads = 4

    order = jnp.argsort(features[:, :, 0], axis=1).astype(jnp.int32)
    inv_order = jnp.argsort(order, axis=1).astype(jnp.int32)
    feat_s = jnp.take_along_axis(features, order[:, :, None], axis=1)
    pos_s = feat_s[:, :, :3]
    pos_col = jnp.pad(pos_s, ((0, 0), (0, 0), (0, 128 - 3)))
    pos_row = jnp.transpose(pos_s, (0, 2, 1))
    pe = _posenc(n, hid)
    b2d = b_conv[None, :]

    rb = min(128, n)
    nblk = n // rb
    jc = min(256, n)
    nc = n // jc
    xs = pos_s[:, :, 0]
    blo = xs[:, ::rb]
    bhi = xs[:, rb - 1::rb]
    clo = xs[:, ::jc]
    chi = xs[:, jc - 1::jc]

    import functools
    smem = pl.BlockSpec(memory_space=pltpu.SMEM)

    x_s = pl.pallas_call(
        functools.partial(_conv_body, jc=jc),
        grid=(t, nblk),
        in_specs=[
            smem, smem, smem, smem,
            pl.BlockSpec((1, rb, 128), lambda ti, bi: (ti, bi, 0)),
            pl.BlockSpec((1, 3, n), lambda ti, bi: (ti, 0, 0)),
            pl.BlockSpec((1, n, in_dim), lambda ti, bi: (ti, 0, 0)),
            pl.BlockSpec((27 * in_dim, hid), lambda ti, bi: (0, 0)),
            pl.BlockSpec((1, hid), lambda ti, bi: (0, 0)),
        ],
        out_specs=pl.BlockSpec((1, rb, hid), lambda ti, bi: (ti, bi, 0)),
        out_shape=jax.ShapeDtypeStruct((t, n, hid), jnp.float32),
        scratch_shapes=[pltpu.VMEM((rb, n), jnp.float32),
                        pltpu.VMEM((rb, 128), jnp.float32),
                        pltpu.VMEM((rb, 27 * in_dim), jnp.float32)],
        compiler_params=pltpu.CompilerParams(
            dimension_semantics=("parallel", "parallel")),
    )(blo, bhi, clo, chi, pos_col, pos_row, feat_s.astype(jnp.bfloat16),
      W_conv.reshape(27 * in_dim, hid).astype(jnp.bfloat16), b2d)

    x = jnp.take_along_axis(x_s, inv_order[:, :, None], axis=1)

    nb = min(256, n)
    nblk2 = n // nb
    wq_t = in_proj_w[:hid].T
    wkv_t = in_proj_w[hid:].T
    bq = in_proj_b[None, :hid]
    bkv = in_proj_b[None, hid:]

    body = functools.partial(_tfm_body, heads=heads)

    y = pl.pallas_call(
        body,
        grid=(nblk2,),
        in_specs=[
            pl.BlockSpec((t, nb, hid), lambda bi: (0, bi, 0)),
            pl.BlockSpec((nb, hid), lambda bi: (bi, 0)),
            pl.BlockSpec((hid, hid), lambda bi: (0, 0)),
            pl.BlockSpec((1, hid), lambda bi: (0, 0)),
            pl.BlockSpec((hid, 2 * hid), lambda bi: (0, 0)),
            pl.BlockSpec((1, 2 * hid), lambda bi: (0, 0)),
            pl.BlockSpec((hid, hid), lambda bi: (0, 0)),
            pl.BlockSpec((1, hid), lambda bi: (0, 0)),
            pl.BlockSpec((1, hid), lambda bi: (0, 0)),
            pl.BlockSpec((1, hid), lambda bi: (0, 0)),
            pl.BlockSpec((hid, 4 * hid), lambda bi: (0, 0)),
            pl.BlockSpec((1, 4 * hid), lambda bi: (0, 0)),
            pl.BlockSpec((4 * hid, hid), lambda bi: (0, 0)),
            pl.BlockSpec((1, hid), lambda bi: (0, 0)),
            pl.BlockSpec((1, hid), lambda bi: (0, 0)),
            pl.BlockSpec((1, hid), lambda bi: (0, 0)),
            pl.BlockSpec((hid, out_dim), lambda bi: (0, 0)),
            pl.BlockSpec((1, out_dim), lambda bi: (0, 0)),
        ],
        out_specs=pl.BlockSpec((nb, out_dim), lambda bi: (bi, 0)),
        out_shape=jax.ShapeDtypeStruct((n, out_dim), jnp.float32),
        compiler_params=pltpu.CompilerParams(
            dimension_semantics=("parallel",)),
    )(x, pe, wq_t, bq, wkv_t, bkv, out_proj_w.T, out_proj_b[None, :],
      ln1_g[None, :], ln1_b[None, :], ff_w1.T, ff_b1[None, :],
      ff_w2.T, ff_b2[None, :], ln2_g[None, :], ln2_b[None, :],
      fc_w.T, fc_b[None, :])

    return y


# speedup vs baseline: 1.1914x; 1.0232x over previous
import math

import functools

import jax
import jax.numpy as jnp
from jax.experimental import pallas as pl
from jax.experimental.pallas import tpu as pltpu
from jax.experimental.pallas import tpu_sc as plsc

_EXTENTS = 0.2
_RADIUS = _EXTENTS / 2.0
_KNN = 64


def _conv_body(blo_ref, bhi_ref, clo_ref, chi_ref,
               pos_col_ref, pos_row_ref, feat_ref, wc_ref, b_ref, out_ref,
               d2_ref, theta_ref, acc_ref, *, jc):
    rb = pos_col_ref.shape[1]
    n = pos_row_ref.shape[2]
    nc = n // jc
    f32 = jnp.float32

    ti = pl.program_id(0)
    bi = pl.program_id(1)

    pc = pos_col_ref[0]
    px_c = pc[:, 0:1]
    py_c = pc[:, 1:2]
    pz_c = pc[:, 2:3]
    pr = pos_row_ref[0]

    r2 = f32(_RADIUS * _RADIUS)
    rm = f32(_RADIUS * (1.0 + 1e-5) + 1e-6)
    my_lo = blo_ref[ti, bi]
    my_hi = bhi_ref[ti, bi]
    hid = wc_ref.shape[1]
    inv_r = f32(1.0 / _RADIUS)
    bf16 = jnp.bfloat16

    def chunk_active(c):
        return jnp.logical_and(clo_ref[ti, c] - my_hi <= rm,
                               my_lo - chi_ref[ti, c] <= rm)

    in_dim = feat_ref.shape[2]

    def accumulate(theta):
        acc_ref[...] = jnp.zeros((rb, 27 * in_dim), dtype=f32)
        theta_ref[:, 1:2] = jnp.zeros((rb, 1), dtype=f32)
        for c in range(nc):
            @pl.when(chunk_active(c))
            def _(c=c):
                sl = slice(c * jc, (c + 1) * jc)
                dx = pr[0:1, sl] - px_c
                dy = pr[1:2, sl] - py_c
                dz = pr[2:3, sl] - pz_c
                d2 = dx * dx + dy * dy + dz * dz
                mask = (d2 <= theta).astype(f32)
                theta_ref[:, 1:2] += jnp.sum(mask, axis=1, keepdims=True)
                q = d2 * f32(1.0 / (_RADIUS * _RADIUS))
                norm2 = jnp.sqrt(q + f32(1e-12))
                ninf = jnp.maximum(jnp.maximum(jnp.abs(dx), jnp.abs(dy)),
                                   jnp.abs(dz)) * inv_r
                scale = jnp.where(ninf > f32(1e-8),
                                  norm2 / jnp.maximum(ninf, f32(1e-8)), f32(0.0))
                scale_d = scale * inv_r

                def axis_w(dv, m):
                    u = jnp.clip(dv * scale_d, f32(-1.0), f32(1.0))
                    whi = jnp.maximum(u, f32(0.0))
                    wlo = whi - u
                    wmid = (f32(1.0) - whi) - wlo
                    if m is not None:
                        wlo, wmid, whi = wlo * m, wmid * m, whi * m
                    return wlo.astype(bf16), wmid.astype(bf16), whi.astype(bf16)

                wx = axis_w(dx, mask)
                wy = axis_w(dy, None)
                wz = axis_w(dz, None)
                feat_c = feat_ref[0, sl, :]
                for a in range(3):
                    for bb in range(3):
                        pab = wx[a] * wy[bb]
                        for cc in range(3):
                            tap = a * 9 + bb * 3 + cc
                            plane = pab * wz[cc]
                            agg = jnp.dot(plane, feat_c,
                                          preferred_element_type=f32)
                            acc_ref[:, tap * in_dim:(tap + 1) * in_dim] += agg

    accumulate(r2)

    @pl.when(jnp.max(theta_ref[:, 1:2]) > f32(_KNN))
    def _cap():
        d2_ref[...] = jnp.ones((rb, n), dtype=f32)
        for c in range(nc):
            @pl.when(chunk_active(c))
            def _(c=c):
                sl = slice(c * jc, (c + 1) * jc)
                dx = pr[0:1, sl] - px_c
                dy = pr[1:2, sl] - py_c
                dz = pr[2:3, sl] - pz_c
                d2_ref[:, sl] = dx * dx + dy * dy + dz * dz
        d2 = d2_ref[...]
        cnt_r = jnp.sum((d2 <= r2).astype(f32), axis=1, keepdims=True)
        lo = jnp.zeros((rb, 1), dtype=f32)
        hi = jnp.full((rb, 1), r2, dtype=f32)
        for _ in range(35):
            mid = (lo + hi) * f32(0.5)
            cm = jnp.sum((d2 <= mid).astype(f32), axis=1, keepdims=True)
            ge = cm >= f32(_KNN)
            hi = jnp.where(ge, mid, hi)
            lo = jnp.where(ge, lo, mid)
        theta_ref[:, 0:1] = jnp.where(cnt_r > f32(_KNN), hi, r2)
        accumulate(theta_ref[:, 0:1])

    cnt = jnp.maximum(theta_ref[:, 1:2], f32(1.0))
    acc = jnp.dot(acc_ref[...].astype(bf16), wc_ref[...],
                  preferred_element_type=f32)
    out = acc / cnt + b_ref[0:1, :]
    out_ref[0] = jnp.maximum(out, f32(0.0))


def _tfm_body(x_ref, pe_ref, wq_ref, bq_ref, wkv_ref, bkv_ref, wo_ref, bo_ref,
              ln1g_ref, ln1b_ref, ffw1_ref, ffb1_ref, ffw2_ref, ffb2_ref,
              ln2g_ref, ln2b_ref, fcw_ref, fcb_ref, out_ref, *, heads):
    f32 = jnp.float32
    t, nb, h = x_ref.shape
    hd = h // heads

    x = x_ref[...] + pe_ref[...]
    x3 = x[t - 1]
    q3 = jnp.dot(x3, wq_ref[...], preferred_element_type=f32) + bq_ref[0:1, :]
    xf = x.reshape(t * nb, h)
    kv = jnp.dot(xf, wkv_ref[...], preferred_element_type=f32) + bkv_ref[0:1, :]
    k = kv[:, :h].reshape(t, nb, h)
    v = kv[:, h:].reshape(t, nb, h)

    ri = jax.lax.broadcasted_iota(jnp.int32, (h, heads), 0)
    ci = jax.lax.broadcasted_iota(jnp.int32, (h, heads), 1)
    seg = (ri // hd == ci).astype(f32)

    inv_sqrt_hd = f32(1.0 / math.sqrt(hd))
    logits = [jnp.dot(q3 * k[t2], seg, preferred_element_type=f32) * inv_sqrt_hd
              for t2 in range(t)]
    m = logits[0]
    for t2 in range(1, t):
        m = jnp.maximum(m, logits[t2])
    exps = [jnp.exp(l - m) for l in logits]
    ssum = exps[0]
    for t2 in range(1, t):
        ssum = ssum + exps[t2]
    o3 = jnp.zeros((nb, h), dtype=f32)
    for t2 in range(t):
        p = exps[t2] / ssum
        pe = jnp.dot(p, seg.T, preferred_element_type=f32)
        o3 = o3 + pe * v[t2]

    a3 = jnp.dot(o3, wo_ref[...], preferred_element_type=f32) + bo_ref[0:1, :]

    def ln(y, g_ref, b_ref):
        mu = jnp.mean(y, axis=1, keepdims=True)
        d = y - mu
        var = jnp.mean(d * d, axis=1, keepdims=True)
        return d * jax.lax.rsqrt(var + f32(1e-5)) * g_ref[0:1, :] + b_ref[0:1, :]

    h1 = ln(x3 + a3, ln1g_ref, ln1b_ref)
    ffh = jnp.maximum(jnp.dot(h1, ffw1_ref[...], preferred_element_type=f32)
                      + ffb1_ref[0:1, :], f32(0.0))
    ff2 = jnp.dot(ffh, ffw2_ref[...], preferred_element_type=f32) + ffb2_ref[0:1, :]
    h2 = ln(h1 + ff2, ln2g_ref, ln2b_ref)
    out_ref[...] = jnp.dot(h2, fcw_ref[...], preferred_element_type=f32) + fcb_ref[0:1, :]


def _sc_row_gather(table, idx):
    info = plsc.get_sparse_core_info()
    nw = info.num_cores * info.num_subcores
    b = idx.shape[0]
    d = table.shape[1]
    b_per_w = b // nw
    mesh = plsc.VectorSubcoreMesh(core_axis_name="c", subcore_axis_name="s")

    @functools.partial(
        pl.kernel, mesh=mesh,
        out_type=jax.ShapeDtypeStruct((b, d), table.dtype),
        scratch_types=[pltpu.VMEM((b_per_w,), jnp.int32),
                       pltpu.VMEM((b_per_w, d), table.dtype),
                       pltpu.SemaphoreType.DMA],
    )
    def k(table_hbm, idx_hbm, out_hbm, idx_v, rows_v, sem):
        wid = jax.lax.axis_index("s") * info.num_cores + jax.lax.axis_index("c")
        base = wid * b_per_w
        pltpu.sync_copy(idx_hbm.at[pl.ds(base, b_per_w)], idx_v)
        pltpu.async_copy(table_hbm.at[idx_v], rows_v, sem).wait()
        pltpu.sync_copy(rows_v, out_hbm.at[pl.ds(base, b_per_w)])

    return k(table, idx)


def _posenc(n, d):
    pos = jnp.arange(n, dtype=jnp.float32)[:, None]
    div = jnp.exp(jnp.arange(0, d, 2, dtype=jnp.float32) * (-math.log(10000.0) / d))
    enc = jnp.zeros((n, d), jnp.float32)
    enc = enc.at[:, 0::2].set(jnp.sin(pos * div))
    enc = enc.at[:, 1::2].set(jnp.cos(pos * div))
    return enc


def kernel(features, W_conv, b_conv, in_proj_w, in_proj_b, out_proj_w, out_proj_b,
           ln1_g, ln1_b, ff_w1, ff_b1, ff_w2, ff_b2, ln2_g, ln2_b, fc_w, fc_b):
    t, n, in_dim = features.shape
    hid = W_conv.shape[2]
    out_dim = fc_w.shape[0]
    heads = 4

    order = jnp.argsort(features[:, :, 0], axis=1).astype(jnp.int32)
    inv_order = jnp.argsort(order, axis=1).astype(jnp.int32)
    feat_s = jnp.take_along_axis(features, order[:, :, None], axis=1)
    pos_s = feat_s[:, :, :3]
    pos_col = jnp.pad(pos_s, ((0, 0), (0, 0), (0, 128 - 3)))
    pos_row = jnp.transpose(pos_s, (0, 2, 1))
    pe = _posenc(n, hid)
    b2d = b_conv[None, :]

    rb = min(128, n)
    nblk = n // rb
    jc = min(256, n)
    nc = n // jc
    xs = pos_s[:, :, 0]
    blo = xs[:, ::rb]
    bhi = xs[:, rb - 1::rb]
    clo = xs[:, ::jc]
    chi = xs[:, jc - 1::jc]

    import functools
    smem = pl.BlockSpec(memory_space=pltpu.SMEM)

    x_s = pl.pallas_call(
        functools.partial(_conv_body, jc=jc),
        grid=(t, nblk),
        in_specs=[
            smem, smem, smem, smem,
            pl.BlockSpec((1, rb, 128), lambda ti, bi: (ti, bi, 0)),
            pl.BlockSpec((1, 3, n), lambda ti, bi: (ti, 0, 0)),
            pl.BlockSpec((1, n, in_dim), lambda ti, bi: (ti, 0, 0)),
            pl.BlockSpec((27 * in_dim, hid), lambda ti, bi: (0, 0)),
            pl.BlockSpec((1, hid), lambda ti, bi: (0, 0)),
        ],
        out_specs=pl.BlockSpec((1, rb, hid), lambda ti, bi: (ti, bi, 0)),
        out_shape=jax.ShapeDtypeStruct((t, n, hid), jnp.float32),
        scratch_shapes=[pltpu.VMEM((rb, n), jnp.float32),
                        pltpu.VMEM((rb, 128), jnp.float32),
                        pltpu.VMEM((rb, 27 * in_dim), jnp.float32)],
        compiler_params=pltpu.CompilerParams(
            dimension_semantics=("parallel", "parallel")),
    )(blo, bhi, clo, chi, pos_col, pos_row, feat_s.astype(jnp.bfloat16),
      W_conv.reshape(27 * in_dim, hid).astype(jnp.bfloat16), b2d)

    flat_idx = (inv_order
                + (jnp.arange(t, dtype=jnp.int32) * n)[:, None]).reshape(-1)
    x = _sc_row_gather(x_s.reshape(t * n, hid), flat_idx).reshape(t, n, hid)

    nb = min(256, n)
    nblk2 = n // nb
    wq_t = in_proj_w[:hid].T
    wkv_t = in_proj_w[hid:].T
    bq = in_proj_b[None, :hid]
    bkv = in_proj_b[None, hid:]

    body = functools.partial(_tfm_body, heads=heads)

    y = pl.pallas_call(
        body,
        grid=(nblk2,),
        in_specs=[
            pl.BlockSpec((t, nb, hid), lambda bi: (0, bi, 0)),
            pl.BlockSpec((nb, hid), lambda bi: (bi, 0)),
            pl.BlockSpec((hid, hid), lambda bi: (0, 0)),
            pl.BlockSpec((1, hid), lambda bi: (0, 0)),
            pl.BlockSpec((hid, 2 * hid), lambda bi: (0, 0)),
            pl.BlockSpec((1, 2 * hid), lambda bi: (0, 0)),
            pl.BlockSpec((hid, hid), lambda bi: (0, 0)),
            pl.BlockSpec((1, hid), lambda bi: (0, 0)),
            pl.BlockSpec((1, hid), lambda bi: (0, 0)),
            pl.BlockSpec((1, hid), lambda bi: (0, 0)),
            pl.BlockSpec((hid, 4 * hid), lambda bi: (0, 0)),
            pl.BlockSpec((1, 4 * hid), lambda bi: (0, 0)),
            pl.BlockSpec((4 * hid, hid), lambda bi: (0, 0)),
            pl.BlockSpec((1, hid), lambda bi: (0, 0)),
            pl.BlockSpec((1, hid), lambda bi: (0, 0)),
            pl.BlockSpec((1, hid), lambda bi: (0, 0)),
            pl.BlockSpec((hid, out_dim), lambda bi: (0, 0)),
            pl.BlockSpec((1, out_dim), lambda bi: (0, 0)),
        ],
        out_specs=pl.BlockSpec((nb, out_dim), lambda bi: (bi, 0)),
        out_shape=jax.ShapeDtypeStruct((n, out_dim), jnp.float32),
        compiler_params=pltpu.CompilerParams(
            dimension_semantics=("parallel",)),
    )(x, pe, wq_t, bq, wkv_t, bkv, out_proj_w.T, out_proj_b[None, :],
      ln1_g[None, :], ln1_b[None, :], ff_w1.T, ff_b1[None, :],
      ff_w2.T, ff_b2[None, :], ln2_g[None, :], ln2_b[None, :],
      fc_w.T, fc_b[None, :])

    return y
